# baseline (speedup 1.0000x reference)
import functools

import jax
import jax.numpy as jnp
from jax import lax
from jax.experimental import pallas as pl
from jax.experimental.pallas import tpu as pltpu
from jax.experimental.pallas import tpu_sc as plsc

_NUM_CORES = 2
_NUM_SUBCORES = 16
_NUM_WORKERS = _NUM_CORES * _NUM_SUBCORES

_CHUNK = 16


@functools.lru_cache(maxsize=None)
def _make_gather(n_total: int, vocab: int, d: int):
  n_per_w = n_total // _NUM_WORKERS
  chunks = n_per_w // _CHUNK
  assert chunks % 2 == 0 and chunks * _CHUNK == n_per_w

  mesh = plsc.VectorSubcoreMesh(core_axis_name="c", subcore_axis_name="s")

  @functools.partial(
      pl.kernel,
      out_type=jax.ShapeDtypeStruct((n_total, d), jnp.float32),
      mesh=mesh,
      scratch_types=[
          pltpu.VMEM((n_per_w,), jnp.int32),
          pltpu.VMEM((_CHUNK, d), jnp.float32),
          pltpu.VMEM((_CHUNK, d), jnp.float32),
          pltpu.SemaphoreType.DMA,
          pltpu.SemaphoreType.DMA,
          pltpu.SemaphoreType.DMA,
          pltpu.SemaphoreType.DMA,
      ],
  )
  def gather_kernel(ids_hbm, table_hbm, out_hbm, idx_v, rows0, rows1,
                    gsem0, gsem1, ssem0, ssem1):
    wid = lax.axis_index("s") * _NUM_CORES + lax.axis_index("c")
    base = wid * n_per_w
    pltpu.sync_copy(ids_hbm.at[pl.ds(base, n_per_w)], idx_v)

    bufs = (rows0, rows1)
    gsems = (gsem0, gsem1)
    ssems = (ssem0, ssem1)

    def start_gather(g, b):
      off = pl.multiple_of(g * _CHUNK, 8)
      pltpu.async_copy(table_hbm.at[idx_v.at[pl.ds(off, _CHUNK)]],
                       bufs[b], gsems[b])

    def wait_gather(b):
      pltpu.make_async_copy(
          table_hbm.at[idx_v.at[pl.ds(0, _CHUNK)]], bufs[b], gsems[b]
      ).wait()

    def start_store(g, b):
      row = pl.multiple_of(base + g * _CHUNK, 8)
      pltpu.async_copy(bufs[b], out_hbm.at[pl.ds(row, _CHUNK)], ssems[b])

    def wait_store(b):
      pltpu.make_async_copy(
          bufs[b], out_hbm.at[pl.ds(base, _CHUNK)], ssems[b]).wait()

    start_gather(0, 0)

    def body(i, carry):
      del carry
      for b in range(2):
        g = i * 2 + b
        nxt = g + 1
        pb = (b + 1) % 2

        wait_gather(b)
        start_store(g, b)

        @pl.when((nxt < chunks) & (g >= 1))
        def _():
          wait_store(pb)

        @pl.when(nxt < chunks)
        def _():
          start_gather(nxt, pb)

      return 0

    lax.fori_loop(0, chunks // 2, body, 0, unroll=1)
    wait_store(0)
    wait_store(1)

  return gather_kernel


def kernel(input_ids, embed_tokens):
  b, s = input_ids.shape
  v, d = embed_tokens.shape
  n = b * s
  flat_ids = input_ids.reshape(n)
  out = _make_gather(n, v, d)(flat_ids, embed_tokens)
  return out.reshape(b, s, d)


# speedup vs baseline: 1.0291x; 1.0291x over previous
import functools

import jax
import jax.numpy as jnp
from jax import lax
from jax.experimental import pallas as pl
from jax.experimental.pallas import tpu as pltpu
from jax.experimental.pallas import tpu_sc as plsc

_NUM_CORES = 2
_NUM_SUBCORES = 16
_NUM_WORKERS = _NUM_CORES * _NUM_SUBCORES

_CHUNK = 8
_NBUF = 4


@functools.lru_cache(maxsize=None)
def _make_gather(n_total: int, vocab: int, d: int):
  n_per_w = n_total // _NUM_WORKERS
  chunks = n_per_w // _CHUNK
  assert chunks % _NBUF == 0 and chunks * _CHUNK == n_per_w

  mesh = plsc.VectorSubcoreMesh(core_axis_name="c", subcore_axis_name="s")

  row_bufs = [pltpu.VMEM((_CHUNK, d), jnp.float32) for _ in range(_NBUF)]
  gsem_types = [pltpu.SemaphoreType.DMA for _ in range(_NBUF)]
  ssem_types = [pltpu.SemaphoreType.DMA for _ in range(_NBUF)]

  @functools.partial(
      pl.kernel,
      out_type=jax.ShapeDtypeStruct((n_total, d), jnp.float32),
      mesh=mesh,
      scratch_types=[pltpu.VMEM((n_per_w,), jnp.int32)]
      + row_bufs + gsem_types + ssem_types,
  )
  def gather_kernel(ids_hbm, table_hbm, out_hbm, idx_v, *scratch):
    bufs = scratch[:_NBUF]
    gsems = scratch[_NBUF:2 * _NBUF]
    ssems = scratch[2 * _NBUF:]

    wid = lax.axis_index("s") * _NUM_CORES + lax.axis_index("c")
    base = wid * n_per_w
    pltpu.sync_copy(ids_hbm.at[pl.ds(base, n_per_w)], idx_v)

    def start_gather(g, b):
      off = pl.multiple_of(g * _CHUNK, 8)
      pltpu.async_copy(table_hbm.at[idx_v.at[pl.ds(off, _CHUNK)]],
                       bufs[b], gsems[b])

    def wait_gather(b):
      pltpu.make_async_copy(
          table_hbm.at[idx_v.at[pl.ds(0, _CHUNK)]], bufs[b], gsems[b]
      ).wait()

    def start_store(g, b):
      row = pl.multiple_of(base + g * _CHUNK, 8)
      pltpu.async_copy(bufs[b], out_hbm.at[pl.ds(row, _CHUNK)], ssems[b])

    def wait_store(b):
      pltpu.make_async_copy(
          bufs[b], out_hbm.at[pl.ds(base, _CHUNK)], ssems[b]).wait()

    for g in range(_NBUF - 1):
      start_gather(g, g)

    def body(i, carry):
      del carry
      for b in range(_NBUF):
        g = i * _NBUF + b
        nxt = g + _NBUF - 1
        pb = (b + _NBUF - 1) % _NBUF

        wait_gather(b)
        start_store(g, b)

        @pl.when((nxt < chunks) & (g >= 1))
        def _():
          wait_store(pb)

        @pl.when(nxt < chunks)
        def _():
          start_gather(nxt, pb)

      return 0

    lax.fori_loop(0, chunks // _NBUF, body, 0, unroll=1)
    for b in range(_NBUF):
      wait_store(b)

  return gather_kernel


def kernel(input_ids, embed_tokens):
  b, s = input_ids.shape
  v, d = embed_tokens.shape
  n = b * s
  flat_ids = input_ids.reshape(n)
  out = _make_gather(n, v, d)(flat_ids, embed_tokens)
  return out.reshape(b, s, d)


# speedup vs baseline: 1.0436x; 1.0141x over previous
import functools

import jax
import jax.numpy as jnp
from jax import lax
from jax.experimental import pallas as pl
from jax.experimental.pallas import tpu as pltpu
from jax.experimental.pallas import tpu_sc as plsc

_NUM_CORES = 2
_NUM_SUBCORES = 16
_NUM_WORKERS = _NUM_CORES * _NUM_SUBCORES

_CHUNK = 8
_NBUF = 7


@functools.lru_cache(maxsize=None)
def _make_gather(n_total: int, vocab: int, d: int):
  n_per_w = n_total // _NUM_WORKERS
  chunks = n_per_w // _CHUNK
  assert chunks * _CHUNK == n_per_w and chunks >= _NBUF

  mesh = plsc.VectorSubcoreMesh(core_axis_name="c", subcore_axis_name="s")

  row_bufs = [pltpu.VMEM((_CHUNK, d), jnp.float32) for _ in range(_NBUF)]
  gsem_types = [pltpu.SemaphoreType.DMA for _ in range(_NBUF)]
  ssem_types = [pltpu.SemaphoreType.DMA for _ in range(_NBUF)]

  @functools.partial(
      pl.kernel,
      out_type=jax.ShapeDtypeStruct((n_total, d), jnp.float32),
      mesh=mesh,
      scratch_types=[pltpu.VMEM((n_per_w,), jnp.int32)]
      + row_bufs + gsem_types + ssem_types,
  )
  def gather_kernel(ids_hbm, table_hbm, out_hbm, idx_v, *scratch):
    bufs = scratch[:_NBUF]
    gsems = scratch[_NBUF:2 * _NBUF]
    ssems = scratch[2 * _NBUF:]

    wid = lax.axis_index("s") * _NUM_CORES + lax.axis_index("c")
    base = wid * n_per_w
    pltpu.sync_copy(ids_hbm.at[pl.ds(base, n_per_w)], idx_v)

    def start_gather(g, b):
      off = pl.multiple_of(g * _CHUNK, 8)
      pltpu.async_copy(table_hbm.at[idx_v.at[pl.ds(off, _CHUNK)]],
                       bufs[b], gsems[b])

    def wait_gather(b):
      pltpu.make_async_copy(
          table_hbm.at[idx_v.at[pl.ds(0, _CHUNK)]], bufs[b], gsems[b]
      ).wait()

    def start_store(g, b):
      row = pl.multiple_of(base + g * _CHUNK, 8)
      pltpu.async_copy(bufs[b], out_hbm.at[pl.ds(row, _CHUNK)], ssems[b])

    def wait_store(b):
      pltpu.make_async_copy(
          bufs[b], out_hbm.at[pl.ds(base, _CHUNK)], ssems[b]).wait()

    for g in range(_NBUF - 1):
      start_gather(g, g)

    n_outer = (chunks + _NBUF - 1) // _NBUF

    def body(i, carry):
      del carry
      for b in range(_NBUF):
        g = i * _NBUF + b
        nxt = g + _NBUF - 1
        pb = (b + _NBUF - 1) % _NBUF

        @pl.when(g < chunks)
        def _():
          wait_gather(b)
          start_store(g, b)

          @pl.when((nxt < chunks) & (g >= 1))
          def _():
            wait_store(pb)

          @pl.when(nxt < chunks)
          def _():
            start_gather(nxt, pb)

      return 0

    lax.fori_loop(0, n_outer, body, 0, unroll=1)
    for b in range(_NBUF):
      wait_store(b)

  return gather_kernel


def kernel(input_ids, embed_tokens):
  b, s = input_ids.shape
  v, d = embed_tokens.shape
  n = b * s
  flat_ids = input_ids.reshape(n)
  out = _make_gather(n, v, d)(flat_ids, embed_tokens)
  return out.reshape(b, s, d)
